# baseline (speedup 1.0000x reference)
import functools

import jax
import jax.numpy as jnp
from jax import lax
from jax.experimental import pallas as pl
from jax.experimental.pallas import tpu as pltpu
from jax.experimental.pallas import tpu_sc as plsc

WORD_DIM = 64
INPUT_DIM = 128
NW = 32
CHUNK = 128


def _sc_gather(idx_hbm, table_hbm, out_hbm, idx_v, rows_v, sem):
    wid = lax.axis_index("s") * 2 + lax.axis_index("c")
    total = out_hbm.shape[0]
    per_w = total // NW
    n_it = per_w // CHUNK
    base = wid * per_w

    def body(i, carry):
        off = base + i * CHUNK
        pltpu.sync_copy(idx_hbm.at[pl.ds(off, CHUNK)], idx_v)
        pltpu.async_copy(table_hbm.at[idx_v], rows_v, sem).wait()
        pltpu.sync_copy(rows_v, out_hbm.at[pl.ds(off, CHUNK)])
        return carry

    lax.fori_loop(0, n_it, body, 0)


def _tc_proj(emb_ref, w_ref, b_ref, out_ref):
    acc = jnp.dot(emb_ref[...], w_ref[...],
                  preferred_element_type=jnp.float32)
    out_ref[...] = jnp.tanh(acc + b_ref[...])


def kernel(word_ids, word_table, W, b):
    Bb, L = word_ids.shape
    total = Bb * L
    flat_ids = word_ids.reshape(total).astype(jnp.int32)

    mesh = plsc.VectorSubcoreMesh(core_axis_name="c", subcore_axis_name="s")
    gather = pl.kernel(
        _sc_gather,
        mesh=mesh,
        compiler_params=pltpu.CompilerParams(use_tc_tiling_on_sc=False),
        out_type=jax.ShapeDtypeStruct((total, WORD_DIM), jnp.float32),
        scratch_types=[
            pltpu.VMEM((CHUNK,), jnp.int32),
            pltpu.VMEM((CHUNK, WORD_DIM), jnp.float32),
            pltpu.SemaphoreType.DMA,
        ],
    )
    emb = gather(flat_ids, word_table)

    rows_blk = 2048
    out = pl.pallas_call(
        _tc_proj,
        grid=(total // rows_blk,),
        in_specs=[
            pl.BlockSpec((rows_blk, WORD_DIM), lambda i: (i, 0)),
            pl.BlockSpec((WORD_DIM, INPUT_DIM), lambda i: (0, 0)),
            pl.BlockSpec((1, INPUT_DIM), lambda i: (0, 0)),
        ],
        out_specs=pl.BlockSpec((rows_blk, INPUT_DIM), lambda i: (i, 0)),
        out_shape=jax.ShapeDtypeStruct((total, INPUT_DIM), jnp.float32),
    )(emb, W, b.reshape(1, INPUT_DIM))

    return out.reshape(Bb, L, INPUT_DIM)


# speedup vs baseline: 2.4513x; 2.4513x over previous
import jax
import jax.numpy as jnp
from jax import lax
from jax.experimental import pallas as pl
from jax.experimental.pallas import tpu as pltpu
from jax.experimental.pallas import tpu_sc as plsc

WORD_DIM = 64
INPUT_DIM = 128
NW = 32
CHUNK = 512
CB = 4096


def _tc_project(tblk_ref, w_ref, b_ref, out_ref):
    acc = lax.dot_general(
        tblk_ref[...], w_ref[...],
        dimension_numbers=(((0,), (0,)), ((), ())),
        preferred_element_type=jnp.float32,
    )
    out_ref[...] = jnp.tanh(acc + b_ref[...])


def _sc_gather(idx_hbm, ptable_hbm, out_hbm, idx_v, rows_v, sem):
    wid = lax.axis_index("s") * 2 + lax.axis_index("c")
    total = out_hbm.shape[0]
    per_w = total // NW
    n_it = per_w // CHUNK
    base = wid * per_w

    def body(i, carry):
        off = base + i * CHUNK
        pltpu.sync_copy(idx_hbm.at[pl.ds(off, CHUNK)], idx_v)
        pltpu.async_copy(ptable_hbm.at[idx_v], rows_v, sem).wait()
        pltpu.sync_copy(rows_v, out_hbm.at[pl.ds(off, CHUNK)])
        return carry

    lax.fori_loop(0, n_it, body, 0)


def kernel(word_ids, word_table, W, b):
    Bb, L = word_ids.shape
    total = Bb * L
    vocab = word_table.shape[0]
    flat_ids = word_ids.reshape(total).astype(jnp.int32)
    tableT = word_table.T

    grid = pl.cdiv(vocab, CB)
    ptable = pl.pallas_call(
        _tc_project,
        grid=(grid,),
        in_specs=[
            pl.BlockSpec((WORD_DIM, CB), lambda i: (0, i)),
            pl.BlockSpec((WORD_DIM, INPUT_DIM), lambda i: (0, 0)),
            pl.BlockSpec((1, INPUT_DIM), lambda i: (0, 0)),
        ],
        out_specs=pl.BlockSpec((CB, INPUT_DIM), lambda i: (i, 0)),
        out_shape=jax.ShapeDtypeStruct((vocab, INPUT_DIM), jnp.float32),
    )(tableT, W, b.reshape(1, INPUT_DIM))

    mesh = plsc.VectorSubcoreMesh(core_axis_name="c", subcore_axis_name="s")
    gather = pl.kernel(
        _sc_gather,
        mesh=mesh,
        out_type=jax.ShapeDtypeStruct((total, INPUT_DIM), jnp.float32),
        scratch_types=[
            pltpu.VMEM((CHUNK,), jnp.int32),
            pltpu.VMEM((CHUNK, INPUT_DIM), jnp.float32),
            pltpu.SemaphoreType.DMA,
        ],
    )
    out = gather(flat_ids, ptable)

    return out.reshape(Bb, L, INPUT_DIM)


# speedup vs baseline: 2.6038x; 1.0622x over previous
import jax
import jax.numpy as jnp
from jax import lax
from jax.experimental import pallas as pl
from jax.experimental.pallas import tpu as pltpu
from jax.experimental.pallas import tpu_sc as plsc

WORD_DIM = 64
INPUT_DIM = 128
NW = 32
CHUNK = 400
CB = 4096


def _tc_project(tblk_ref, w_ref, b_ref, out_ref):
    acc = lax.dot_general(
        tblk_ref[...], w_ref[...],
        dimension_numbers=(((0,), (0,)), ((), ())),
        preferred_element_type=jnp.float32,
    )
    out_ref[...] = jnp.tanh(acc + b_ref[...])


def _sc_gather(idx_hbm, ptable_hbm, out_hbm,
               idx0, idx1, rows0, rows1, sem0, sem1):
    wid = lax.axis_index("s") * 2 + lax.axis_index("c")
    total = out_hbm.shape[0]
    per_w = total // NW
    n_it = per_w // CHUNK
    base = wid * per_w
    idx = (idx0, idx1)
    rows = (rows0, rows1)
    sem = (sem0, sem1)

    def issue(i, p):
        pltpu.sync_copy(idx_hbm.at[pl.ds(base + i * CHUNK, CHUNK)], idx[p])
        pltpu.async_copy(ptable_hbm.at[idx[p]], rows[p], sem[p])

    issue(0, 0)

    def body(j, carry):
        for p in range(2):
            i = 2 * j + p

            @pl.when(i + 1 < n_it)
            def _():
                issue(i + 1, 1 - p)

            pltpu.make_async_copy(ptable_hbm.at[idx[p]], rows[p], sem[p]).wait()
            pltpu.sync_copy(rows[p], out_hbm.at[pl.ds(base + i * CHUNK, CHUNK)])
        return carry

    lax.fori_loop(0, n_it // 2, body, 0)


def kernel(word_ids, word_table, W, b):
    Bb, L = word_ids.shape
    total = Bb * L
    vocab = word_table.shape[0]
    flat_ids = word_ids.reshape(total).astype(jnp.int32)
    tableT = word_table.T

    grid = pl.cdiv(vocab, CB)
    ptable = pl.pallas_call(
        _tc_project,
        grid=(grid,),
        in_specs=[
            pl.BlockSpec((WORD_DIM, CB), lambda i: (0, i)),
            pl.BlockSpec((WORD_DIM, INPUT_DIM), lambda i: (0, 0)),
            pl.BlockSpec((1, INPUT_DIM), lambda i: (0, 0)),
        ],
        out_specs=pl.BlockSpec((CB, INPUT_DIM), lambda i: (i, 0)),
        out_shape=jax.ShapeDtypeStruct((vocab, INPUT_DIM), jnp.float32),
    )(tableT, W, b.reshape(1, INPUT_DIM))

    mesh = plsc.VectorSubcoreMesh(core_axis_name="c", subcore_axis_name="s")
    gather = pl.kernel(
        _sc_gather,
        mesh=mesh,
        out_type=jax.ShapeDtypeStruct((total, INPUT_DIM), jnp.float32),
        scratch_types=[
            pltpu.VMEM((CHUNK,), jnp.int32),
            pltpu.VMEM((CHUNK,), jnp.int32),
            pltpu.VMEM((CHUNK, INPUT_DIM), jnp.float32),
            pltpu.VMEM((CHUNK, INPUT_DIM), jnp.float32),
            pltpu.SemaphoreType.DMA,
            pltpu.SemaphoreType.DMA,
        ],
    )
    out = gather(flat_ids, ptable)

    return out.reshape(Bb, L, INPUT_DIM)


# speedup vs baseline: 2.9517x; 1.1336x over previous
import jax
import jax.numpy as jnp
from jax import lax
from jax.experimental import pallas as pl
from jax.experimental.pallas import tpu as pltpu
from jax.experimental.pallas import tpu_sc as plsc

WORD_DIM = 64
INPUT_DIM = 128
NW = 32
CHUNK = 400
CB = 8192


def _tc_project(tblk_ref, w_ref, b_ref, out_ref):
    acc = lax.dot_general(
        tblk_ref[...], w_ref[...],
        dimension_numbers=(((0,), (0,)), ((), ())),
        preferred_element_type=jnp.float32,
    )
    out_ref[...] = jnp.tanh(acc + b_ref[...])


def _sc_gather(idx_hbm, ptable_hbm, out_hbm,
               idx0, idx1, rows0, rows1, sem0, sem1):
    wid = lax.axis_index("s") * 2 + lax.axis_index("c")
    total = out_hbm.shape[0]
    per_w = total // NW
    n_it = per_w // CHUNK
    base = wid * per_w
    idx = (idx0, idx1)
    rows = (rows0, rows1)
    sem = (sem0, sem1)

    def issue(i, p):
        pltpu.sync_copy(idx_hbm.at[pl.ds(base + i * CHUNK, CHUNK)], idx[p])
        pltpu.async_copy(ptable_hbm.at[idx[p]], rows[p], sem[p])

    issue(0, 0)

    def body(j, carry):
        for p in range(2):
            i = 2 * j + p

            @pl.when(i + 1 < n_it)
            def _():
                issue(i + 1, 1 - p)

            pltpu.make_async_copy(ptable_hbm.at[idx[p]], rows[p], sem[p]).wait()
            pltpu.sync_copy(rows[p], out_hbm.at[pl.ds(base + i * CHUNK, CHUNK)])
        return carry

    lax.fori_loop(0, n_it // 2, body, 0)


def kernel(word_ids, word_table, W, b):
    Bb, L = word_ids.shape
    total = Bb * L
    vocab = word_table.shape[0]
    flat_ids = word_ids.reshape(total).astype(jnp.int32)
    tableT = word_table.T

    grid = pl.cdiv(vocab, CB)
    ptable = pl.pallas_call(
        _tc_project,
        grid=(grid,),
        in_specs=[
            pl.BlockSpec((WORD_DIM, CB), lambda i: (0, i)),
            pl.BlockSpec((WORD_DIM, INPUT_DIM), lambda i: (0, 0)),
            pl.BlockSpec((1, INPUT_DIM), lambda i: (0, 0)),
        ],
        out_specs=pl.BlockSpec((CB, INPUT_DIM), lambda i: (i, 0)),
        out_shape=jax.ShapeDtypeStruct((vocab, INPUT_DIM), jnp.float32),
    )(tableT, W, b.reshape(1, INPUT_DIM))

    mesh = plsc.VectorSubcoreMesh(core_axis_name="c", subcore_axis_name="s")
    gather = pl.kernel(
        _sc_gather,
        mesh=mesh,
        out_type=jax.ShapeDtypeStruct((total, INPUT_DIM), jnp.float32),
        scratch_types=[
            pltpu.VMEM((CHUNK,), jnp.int32),
            pltpu.VMEM((CHUNK,), jnp.int32),
            pltpu.VMEM((CHUNK, INPUT_DIM), jnp.float32),
            pltpu.VMEM((CHUNK, INPUT_DIM), jnp.float32),
            pltpu.SemaphoreType.DMA,
            pltpu.SemaphoreType.DMA,
        ],
    )
    out = gather(flat_ids, ptable)

    return out.reshape(Bb, L, INPUT_DIM)


# speedup vs baseline: 3.0889x; 1.0465x over previous
import jax
import jax.numpy as jnp
from jax import lax
from jax.experimental import pallas as pl
from jax.experimental.pallas import tpu as pltpu
from jax.experimental.pallas import tpu_sc as plsc

WORD_DIM = 64
INPUT_DIM = 128
NW = 32
CHUNK = 400
CB = 16384


def _tc_project(tblk_ref, w_ref, b_ref, out_ref):
    acc = lax.dot_general(
        tblk_ref[...], w_ref[...],
        dimension_numbers=(((0,), (0,)), ((), ())),
        preferred_element_type=jnp.float32,
    )
    out_ref[...] = jnp.tanh(acc + b_ref[...])


def _sc_gather(idx_hbm, ptable_hbm, out_hbm,
               idx0, idx1, rows0, rows1, sem0, sem1):
    wid = lax.axis_index("s") * 2 + lax.axis_index("c")
    total = out_hbm.shape[0]
    per_w = total // NW
    n_it = per_w // CHUNK
    base = wid * per_w
    idx = (idx0, idx1)
    rows = (rows0, rows1)
    sem = (sem0, sem1)

    def issue(i, p):
        pltpu.sync_copy(idx_hbm.at[pl.ds(base + i * CHUNK, CHUNK)], idx[p])
        pltpu.async_copy(ptable_hbm.at[idx[p]], rows[p], sem[p])

    issue(0, 0)

    def body(j, carry):
        for p in range(2):
            i = 2 * j + p

            @pl.when(i + 1 < n_it)
            def _():
                issue(i + 1, 1 - p)

            pltpu.make_async_copy(ptable_hbm.at[idx[p]], rows[p], sem[p]).wait()
            pltpu.sync_copy(rows[p], out_hbm.at[pl.ds(base + i * CHUNK, CHUNK)])
        return carry

    lax.fori_loop(0, n_it // 2, body, 0)


def kernel(word_ids, word_table, W, b):
    Bb, L = word_ids.shape
    total = Bb * L
    vocab = word_table.shape[0]
    flat_ids = word_ids.reshape(total).astype(jnp.int32)
    tableT = word_table.T

    grid = pl.cdiv(vocab, CB)
    ptable = pl.pallas_call(
        _tc_project,
        grid=(grid,),
        in_specs=[
            pl.BlockSpec((WORD_DIM, CB), lambda i: (0, i)),
            pl.BlockSpec((WORD_DIM, INPUT_DIM), lambda i: (0, 0)),
            pl.BlockSpec((1, INPUT_DIM), lambda i: (0, 0)),
        ],
        out_specs=pl.BlockSpec((CB, INPUT_DIM), lambda i: (i, 0)),
        out_shape=jax.ShapeDtypeStruct((vocab, INPUT_DIM), jnp.float32),
    )(tableT, W, b.reshape(1, INPUT_DIM))

    mesh = plsc.VectorSubcoreMesh(core_axis_name="c", subcore_axis_name="s")
    gather = pl.kernel(
        _sc_gather,
        mesh=mesh,
        out_type=jax.ShapeDtypeStruct((total, INPUT_DIM), jnp.float32),
        scratch_types=[
            pltpu.VMEM((CHUNK,), jnp.int32),
            pltpu.VMEM((CHUNK,), jnp.int32),
            pltpu.VMEM((CHUNK, INPUT_DIM), jnp.float32),
            pltpu.VMEM((CHUNK, INPUT_DIM), jnp.float32),
            pltpu.SemaphoreType.DMA,
            pltpu.SemaphoreType.DMA,
        ],
    )
    out = gather(flat_ids, ptable)

    return out.reshape(Bb, L, INPUT_DIM)


# speedup vs baseline: 3.1151x; 1.0085x over previous
import jax
import jax.numpy as jnp
from jax import lax
from jax.experimental import pallas as pl
from jax.experimental.pallas import tpu as pltpu
from jax.experimental.pallas import tpu_sc as plsc

WORD_DIM = 64
INPUT_DIM = 128
NW = 32
CHUNK = 400
CB = 32768


def _tc_project(tblk_ref, w_ref, b_ref, out_ref):
    acc = lax.dot_general(
        tblk_ref[...], w_ref[...],
        dimension_numbers=(((0,), (0,)), ((), ())),
        preferred_element_type=jnp.float32,
    )
    out_ref[...] = jnp.tanh(acc + b_ref[...])


def _sc_gather(idx_hbm, ptable_hbm, out_hbm,
               idx0, idx1, rows0, rows1, sem0, sem1):
    wid = lax.axis_index("s") * 2 + lax.axis_index("c")
    total = out_hbm.shape[0]
    per_w = total // NW
    n_it = per_w // CHUNK
    base = wid * per_w
    idx = (idx0, idx1)
    rows = (rows0, rows1)
    sem = (sem0, sem1)

    def issue(i, p):
        pltpu.sync_copy(idx_hbm.at[pl.ds(base + i * CHUNK, CHUNK)], idx[p])
        pltpu.async_copy(ptable_hbm.at[idx[p]], rows[p], sem[p])

    issue(0, 0)

    def body(j, carry):
        for p in range(2):
            i = 2 * j + p

            @pl.when(i + 1 < n_it)
            def _():
                issue(i + 1, 1 - p)

            pltpu.make_async_copy(ptable_hbm.at[idx[p]], rows[p], sem[p]).wait()
            pltpu.sync_copy(rows[p], out_hbm.at[pl.ds(base + i * CHUNK, CHUNK)])
        return carry

    lax.fori_loop(0, n_it // 2, body, 0)


def kernel(word_ids, word_table, W, b):
    Bb, L = word_ids.shape
    total = Bb * L
    vocab = word_table.shape[0]
    flat_ids = word_ids.reshape(total).astype(jnp.int32)
    tableT = word_table.T

    grid = pl.cdiv(vocab, CB)
    ptable = pl.pallas_call(
        _tc_project,
        grid=(grid,),
        in_specs=[
            pl.BlockSpec((WORD_DIM, CB), lambda i: (0, i)),
            pl.BlockSpec((WORD_DIM, INPUT_DIM), lambda i: (0, 0)),
            pl.BlockSpec((1, INPUT_DIM), lambda i: (0, 0)),
        ],
        out_specs=pl.BlockSpec((CB, INPUT_DIM), lambda i: (i, 0)),
        out_shape=jax.ShapeDtypeStruct((vocab, INPUT_DIM), jnp.float32),
    )(tableT, W, b.reshape(1, INPUT_DIM))

    mesh = plsc.VectorSubcoreMesh(core_axis_name="c", subcore_axis_name="s")
    gather = pl.kernel(
        _sc_gather,
        mesh=mesh,
        out_type=jax.ShapeDtypeStruct((total, INPUT_DIM), jnp.float32),
        scratch_types=[
            pltpu.VMEM((CHUNK,), jnp.int32),
            pltpu.VMEM((CHUNK,), jnp.int32),
            pltpu.VMEM((CHUNK, INPUT_DIM), jnp.float32),
            pltpu.VMEM((CHUNK, INPUT_DIM), jnp.float32),
            pltpu.SemaphoreType.DMA,
            pltpu.SemaphoreType.DMA,
        ],
    )
    out = gather(flat_ids, ptable)

    return out.reshape(Bb, L, INPUT_DIM)


# speedup vs baseline: 3.1190x; 1.0013x over previous
import jax
import jax.numpy as jnp
from jax import lax
from jax.experimental import pallas as pl
from jax.experimental.pallas import tpu as pltpu
from jax.experimental.pallas import tpu_sc as plsc

WORD_DIM = 64
INPUT_DIM = 128
NW = 32
CHUNK = 400
CB = 32768


def _tc_project(tblk_ref, w_ref, b_ref, out_ref):
    acc = lax.dot_general(
        tblk_ref[...], w_ref[...],
        dimension_numbers=(((0,), (0,)), ((), ())),
        preferred_element_type=jnp.float32,
    )
    out_ref[...] = jnp.tanh(acc + b_ref[...])


def _sc_gather(idx_hbm, ptable_hbm, out_hbm,
               idx_all, rows0, rows1, sem0, sem1):
    wid = lax.axis_index("s") * 2 + lax.axis_index("c")
    total = out_hbm.shape[0]
    per_w = total // NW
    n_it = per_w // CHUNK
    base = wid * per_w
    rows = (rows0, rows1)
    sem = (sem0, sem1)

    pltpu.sync_copy(idx_hbm.at[pl.ds(base, per_w)], idx_all)

    def issue(i, p):
        idx = idx_all.at[pl.ds(i * CHUNK, CHUNK)]
        pltpu.async_copy(ptable_hbm.at[idx], rows[p], sem[p])

    issue(0, 0)

    def body(j, carry):
        for p in range(2):
            i = 2 * j + p

            @pl.when(i + 1 < n_it)
            def _():
                issue(i + 1, 1 - p)

            idx = idx_all.at[pl.ds(i * CHUNK, CHUNK)]
            pltpu.make_async_copy(ptable_hbm.at[idx], rows[p], sem[p]).wait()
            pltpu.sync_copy(rows[p], out_hbm.at[pl.ds(base + i * CHUNK, CHUNK)])
        return carry

    lax.fori_loop(0, n_it // 2, body, 0)


def kernel(word_ids, word_table, W, b):
    Bb, L = word_ids.shape
    total = Bb * L
    vocab = word_table.shape[0]
    flat_ids = word_ids.reshape(total).astype(jnp.int32)
    tableT = word_table.T

    grid = pl.cdiv(vocab, CB)
    ptable = pl.pallas_call(
        _tc_project,
        grid=(grid,),
        in_specs=[
            pl.BlockSpec((WORD_DIM, CB), lambda i: (0, i)),
            pl.BlockSpec((WORD_DIM, INPUT_DIM), lambda i: (0, 0)),
            pl.BlockSpec((1, INPUT_DIM), lambda i: (0, 0)),
        ],
        out_specs=pl.BlockSpec((CB, INPUT_DIM), lambda i: (i, 0)),
        out_shape=jax.ShapeDtypeStruct((vocab, INPUT_DIM), jnp.float32),
    )(tableT, W, b.reshape(1, INPUT_DIM))

    mesh = plsc.VectorSubcoreMesh(core_axis_name="c", subcore_axis_name="s")
    gather = pl.kernel(
        _sc_gather,
        mesh=mesh,
        out_type=jax.ShapeDtypeStruct((total, INPUT_DIM), jnp.float32),
        scratch_types=[
            pltpu.VMEM((total // NW,), jnp.int32),
            pltpu.VMEM((CHUNK, INPUT_DIM), jnp.float32),
            pltpu.VMEM((CHUNK, INPUT_DIM), jnp.float32),
            pltpu.SemaphoreType.DMA,
            pltpu.SemaphoreType.DMA,
        ],
    )
    out = gather(flat_ids, ptable)

    return out.reshape(Bb, L, INPUT_DIM)


# speedup vs baseline: 3.1202x; 1.0004x over previous
import jax
import jax.numpy as jnp
from jax import lax
from jax.experimental import pallas as pl
from jax.experimental.pallas import tpu as pltpu
from jax.experimental.pallas import tpu_sc as plsc

WORD_DIM = 64
INPUT_DIM = 128
NW = 32
CHUNK = 200
NBUF = 4
CB = 32768


def _tc_project(tblk_ref, w_ref, b_ref, out_ref):
    acc = lax.dot_general(
        tblk_ref[...], w_ref[...],
        dimension_numbers=(((0,), (0,)), ((), ())),
        preferred_element_type=jnp.float32,
    )
    out_ref[...] = jnp.tanh(acc + b_ref[...])


def _sc_gather(idx_hbm, ptable_hbm, out_hbm,
               idx_all, rows0, rows1, rows2, rows3,
               gsem0, gsem1, gsem2, gsem3, wsem0, wsem1, wsem2, wsem3):
    wid = lax.axis_index("s") * 2 + lax.axis_index("c")
    total = out_hbm.shape[0]
    per_w = total // NW
    n_it = per_w // CHUNK
    base = wid * per_w
    rows = (rows0, rows1, rows2, rows3)
    gsem = (gsem0, gsem1, gsem2, gsem3)
    wsem = (wsem0, wsem1, wsem2, wsem3)

    pltpu.sync_copy(idx_hbm.at[pl.ds(base, per_w)], idx_all)

    def gather_copy(i, p):
        idx = idx_all.at[pl.ds(i * CHUNK, CHUNK)]
        return pltpu.make_async_copy(ptable_hbm.at[idx], rows[p], gsem[p])

    def write_copy(i, p):
        dst = out_hbm.at[pl.ds(base + i * CHUNK, CHUNK)]
        return pltpu.make_async_copy(rows[p], dst, wsem[p])

    gather_copy(0, 0).start()
    gather_copy(1, 1).start()

    def body(j, carry):
        for p in range(NBUF):
            i = NBUF * j + p
            gather_copy(i, p).wait()
            write_copy(i, p).start()
            q = (p + 2) % NBUF

            @pl.when((i + 2 < n_it) & (i >= 2))
            def _():
                write_copy(i - 2, q).wait()
                gather_copy(i + 2, q).start()

            @pl.when((i + 2 < n_it) & (i < 2))
            def _():
                gather_copy(i + 2, q).start()
        return carry

    lax.fori_loop(0, n_it // NBUF, body, 0)
    write_copy(n_it - 2, (n_it - 2) % NBUF).wait()
    write_copy(n_it - 1, (n_it - 1) % NBUF).wait()


def kernel(word_ids, word_table, W, b):
    Bb, L = word_ids.shape
    total = Bb * L
    vocab = word_table.shape[0]
    flat_ids = word_ids.reshape(total).astype(jnp.int32)
    tableT = word_table.T

    grid = pl.cdiv(vocab, CB)
    ptable = pl.pallas_call(
        _tc_project,
        grid=(grid,),
        in_specs=[
            pl.BlockSpec((WORD_DIM, CB), lambda i: (0, i)),
            pl.BlockSpec((WORD_DIM, INPUT_DIM), lambda i: (0, 0)),
            pl.BlockSpec((1, INPUT_DIM), lambda i: (0, 0)),
        ],
        out_specs=pl.BlockSpec((CB, INPUT_DIM), lambda i: (i, 0)),
        out_shape=jax.ShapeDtypeStruct((vocab, INPUT_DIM), jnp.float32),
    )(tableT, W, b.reshape(1, INPUT_DIM))

    mesh = plsc.VectorSubcoreMesh(core_axis_name="c", subcore_axis_name="s")
    gather = pl.kernel(
        _sc_gather,
        mesh=mesh,
        out_type=jax.ShapeDtypeStruct((total, INPUT_DIM), jnp.float32),
        scratch_types=(
            [pltpu.VMEM((total // NW,), jnp.int32)]
            + [pltpu.VMEM((CHUNK, INPUT_DIM), jnp.float32)] * NBUF
            + [pltpu.SemaphoreType.DMA] * (2 * NBUF)
        ),
    )
    out = gather(flat_ids, ptable)

    return out.reshape(Bb, L, INPUT_DIM)
